# baseline (speedup 1.0000x reference)
import jax
import jax.numpy as jnp
import numpy as np
from jax import lax
from jax.experimental import pallas as pl
from jax.experimental.pallas import tpu as pltpu

_N_BINS = 15
_N = 524288
_C = 100
_BLK = 4096
_GRID = _N // _BLK

_bounds = np.linspace(0.0, 1.0, _N_BINS + 1, dtype=np.float32)
_BNDS = np.zeros((16, 128), dtype=np.float32)
_BNDS[:, 0] = 2.0
_BNDS[:, 1] = 3.0
_BNDS[:_N_BINS, 0] = _bounds[:-1]
_BNDS[0, 0] -= 1e-6
_BNDS[:_N_BINS, 1] = _bounds[1:]
_BNDS[:, 2] = np.arange(16, dtype=np.float32)


def _ece_kernel(x_ref, lbl_ref, bnd_ref, out_ref, acc_ref):
    i = pl.program_id(0)

    @pl.when(i == 0)
    def _init():
        acc_ref[...] = jnp.zeros_like(acc_ref)

    x = x_ref[...]
    m = jnp.max(x, axis=1, keepdims=True)
    z = jnp.sum(jnp.exp(x - m), axis=1, keepdims=True)
    conf_col = 1.0 / z
    pred_col = jnp.argmax(x, axis=1, keepdims=True)

    conf = lax.transpose(conf_col, (1, 0))
    pred = lax.transpose(pred_col, (1, 0))
    lbl = lbl_ref[...].reshape(1, _BLK)
    hit = (pred == lbl).astype(jnp.float32)

    t = (15.0 - conf * 15.0).astype(jnp.int32)
    idx = jnp.maximum(jnp.minimum(14 - t, 14), 0)
    idxw = jnp.broadcast_to(idx, (16, _BLK))
    binw = lax.broadcasted_iota(jnp.int32, (16, _BLK), 0)
    maskf = (idxw == binw).astype(jnp.float32)
    ones = jnp.ones((1, _BLK), jnp.float32)
    rhs = jnp.concatenate([ones, conf, hit], axis=0)
    stats = lax.dot_general(
        maskf, rhs, (((1,), (1,)), ((), ())),
        preferred_element_type=jnp.float32,
    )
    acc_ref[:, 0:3] += stats

    @pl.when(i == _GRID - 1)
    def _finish():
        tot = acc_ref[:, 0:1]
        sc = acc_ref[:, 1:2]
        sa = acc_ref[:, 2:3]
        safe = jnp.maximum(tot, 1.0)
        contrib = jnp.abs(sc / safe - sa / safe) * (tot / float(_N))
        contrib = jnp.where(tot > 0.0, contrib, 0.0)
        out_ref[...] = jnp.sum(contrib, axis=0, keepdims=True)


@jax.jit
def kernel(logits_input, labels_input):
    labels = labels_input.astype(jnp.int32).reshape(_GRID, 1, _BLK)
    out = pl.pallas_call(
        _ece_kernel,
        grid=(_GRID,),
        in_specs=[
            pl.BlockSpec((_BLK, _C), lambda i: (i, 0)),
            pl.BlockSpec((1, 1, _BLK), lambda i: (i, 0, 0)),
            pl.BlockSpec((16, 128), lambda i: (0, 0)),
        ],
        out_specs=pl.BlockSpec((1, 1), lambda i: (0, 0)),
        out_shape=jax.ShapeDtypeStruct((1, 1), jnp.float32),
        scratch_shapes=[pltpu.VMEM((16, 128), jnp.float32)],
        compiler_params=pltpu.CompilerParams(
            dimension_semantics=("arbitrary",),
        ),
    )(logits_input, labels, jnp.asarray(_BNDS))
    return out.reshape((1,))


# speedup vs baseline: 1.0118x; 1.0118x over previous
import jax
import jax.numpy as jnp
import numpy as np
from jax import lax
from jax.experimental import pallas as pl
from jax.experimental.pallas import tpu as pltpu

_N_BINS = 15
_N = 524288
_C = 100
_BLK = 8192
_GRID = _N // _BLK

_HBLK = 8
_HGRID = _GRID // _HBLK

_bounds = np.linspace(0.0, 1.0, _N_BINS + 1, dtype=np.float32)
_LOWERS = [float(v) for v in _bounds[:-1]]
_LOWERS[0] = float(np.float32(_bounds[0] - 1e-6))
_UPPERS = [float(v) for v in _bounds[1:]]


def _dense_kernel(x_ref, lbl_ref, conf_ref, hit_ref):
    r = lax.rem(pl.program_id(0), 8)
    x = x_ref[...]
    m = jnp.max(x, axis=1, keepdims=True)
    z = jnp.sum(jnp.exp(x - m), axis=1, keepdims=True)
    conf_col = 1.0 / z
    pred_col = jnp.argmax(x, axis=1, keepdims=True)
    conf = lax.transpose(conf_col, (1, 0))
    pred = lax.transpose(pred_col, (1, 0))
    lbl = lbl_ref[...].reshape(1, _BLK)
    hit = (pred == lbl).astype(jnp.float32)
    conf_ref[pl.ds(r, 1), :] = conf
    hit_ref[pl.ds(r, 1), :] = hit


def _hist_kernel(conf_ref, hit_ref, out_ref, acc_ref):
    i = pl.program_id(0)

    @pl.when(i == 0)
    def _init():
        for b in range(_N_BINS):
            for k in range(3):
                acc_ref[b, k] = 0.0

    c = conf_ref[...]
    h = hit_ref[...]
    for b in range(_N_BINS):
        mask = ((c > _LOWERS[b]) & (c <= _UPPERS[b])).astype(jnp.float32)
        acc_ref[b, 0] += jnp.sum(mask)
        acc_ref[b, 1] += jnp.sum(mask * c)
        acc_ref[b, 2] += jnp.sum(mask * h)

    @pl.when(i == _HGRID - 1)
    def _finish():
        ece = 0.0
        for b in range(_N_BINS):
            cnt = acc_ref[b, 0]
            safe = jnp.maximum(cnt, 1.0)
            contrib = (
                jnp.abs(acc_ref[b, 1] / safe - acc_ref[b, 2] / safe)
                * (cnt / float(_N))
            )
            ece = ece + jnp.where(cnt > 0.0, contrib, 0.0)
        out_ref[...] = jnp.full((1, 1), ece, jnp.float32)


@jax.jit
def kernel(logits_input, labels_input):
    labels = labels_input.astype(jnp.int32).reshape(_GRID, 1, _BLK)
    conf_rows, hit_rows = pl.pallas_call(
        _dense_kernel,
        grid=(_GRID,),
        in_specs=[
            pl.BlockSpec((_BLK, _C), lambda i: (i, 0)),
            pl.BlockSpec((1, 1, _BLK), lambda i: (i, 0, 0)),
        ],
        out_specs=[
            pl.BlockSpec((8, _BLK), lambda i: (i // 8, 0)),
            pl.BlockSpec((8, _BLK), lambda i: (i // 8, 0)),
        ],
        out_shape=[
            jax.ShapeDtypeStruct((_GRID, _BLK), jnp.float32),
            jax.ShapeDtypeStruct((_GRID, _BLK), jnp.float32),
        ],
        compiler_params=pltpu.CompilerParams(
            dimension_semantics=("arbitrary",),
        ),
    )(logits_input, labels)

    out = pl.pallas_call(
        _hist_kernel,
        grid=(_HGRID,),
        in_specs=[
            pl.BlockSpec((_HBLK, _BLK), lambda i: (i, 0)),
            pl.BlockSpec((_HBLK, _BLK), lambda i: (i, 0)),
        ],
        out_specs=pl.BlockSpec((1, 1), lambda i: (0, 0)),
        out_shape=jax.ShapeDtypeStruct((1, 1), jnp.float32),
        scratch_shapes=[pltpu.SMEM((_N_BINS, 3), jnp.float32)],
        compiler_params=pltpu.CompilerParams(
            dimension_semantics=("arbitrary",),
        ),
    )(conf_rows, hit_rows)
    return out.reshape((1,))


# speedup vs baseline: 1.0202x; 1.0083x over previous
import jax
import jax.numpy as jnp
import numpy as np
from jax import lax
from jax.experimental import pallas as pl
from jax.experimental.pallas import tpu as pltpu

_N_BINS = 15
_N = 524288
_C = 100
_BLK = 8192
_GRID = _N // _BLK

_bounds = np.linspace(0.0, 1.0, _N_BINS + 1, dtype=np.float32)
_BNDS = np.zeros((16, 128), dtype=np.float32)
_BNDS[:, 0] = 2.0
_BNDS[:, 1] = 3.0
_BNDS[:_N_BINS, 0] = _bounds[:-1]
_BNDS[0, 0] -= 1e-6
_BNDS[:_N_BINS, 1] = _bounds[1:]


def _ece_kernel(x_ref, lbl_ref, bnd_ref, out_ref, acc_ref):
    i = pl.program_id(0)

    @pl.when(i == 0)
    def _init():
        acc_ref[...] = jnp.zeros_like(acc_ref)

    x = x_ref[...]
    m = jnp.max(x, axis=1, keepdims=True)
    z = jnp.sum(jnp.exp(x - m), axis=1, keepdims=True)
    conf_col = 1.0 / z
    pred_col = jnp.argmax(x, axis=1, keepdims=True)

    conf = lax.transpose(conf_col, (1, 0))
    pred = lax.transpose(pred_col, (1, 0))
    lbl = lbl_ref[...].reshape(1, _BLK)
    hit = (pred == lbl).astype(jnp.float32)

    lo = bnd_ref[:, 0:1]
    up = bnd_ref[:, 1:2]
    maskf = ((conf > lo) & (conf <= up)).astype(jnp.float32)
    cnt = jnp.sum(maskf, axis=1, keepdims=True)
    sconf = jnp.sum(maskf * conf, axis=1, keepdims=True)
    sacc = jnp.sum(maskf * hit, axis=1, keepdims=True)
    acc_ref[:, 0:1] += cnt
    acc_ref[:, 1:2] += sconf
    acc_ref[:, 2:3] += sacc

    @pl.when(i == _GRID - 1)
    def _finish():
        tot = acc_ref[:, 0:1]
        sc = acc_ref[:, 1:2]
        sa = acc_ref[:, 2:3]
        safe = jnp.maximum(tot, 1.0)
        contrib = jnp.abs(sc / safe - sa / safe) * (tot / float(_N))
        contrib = jnp.where(tot > 0.0, contrib, 0.0)
        out_ref[...] = jnp.sum(contrib, axis=0, keepdims=True)


@jax.jit
def kernel(logits_input, labels_input):
    labels = labels_input.astype(jnp.int32).reshape(_GRID, 1, _BLK)
    out = pl.pallas_call(
        _ece_kernel,
        grid=(_GRID,),
        in_specs=[
            pl.BlockSpec((_BLK, _C), lambda i: (i, 0)),
            pl.BlockSpec((1, 1, _BLK), lambda i: (i, 0, 0)),
            pl.BlockSpec((16, 128), lambda i: (0, 0)),
        ],
        out_specs=pl.BlockSpec((1, 1), lambda i: (0, 0)),
        out_shape=jax.ShapeDtypeStruct((1, 1), jnp.float32),
        scratch_shapes=[pltpu.VMEM((16, 128), jnp.float32)],
        compiler_params=pltpu.CompilerParams(
            dimension_semantics=("arbitrary",),
        ),
    )(logits_input, labels, jnp.asarray(_BNDS))
    return out.reshape((1,))


# speedup vs baseline: 1.0304x; 1.0100x over previous
import jax
import jax.numpy as jnp
import numpy as np
from jax import lax
from jax.experimental import pallas as pl
from jax.experimental.pallas import tpu as pltpu

_N_BINS = 15
_N = 524288
_C = 100
_BLK = 16384
_GRID = _N // _BLK

_bounds = np.linspace(0.0, 1.0, _N_BINS + 1, dtype=np.float32)
_BNDS = np.zeros((16, 128), dtype=np.float32)
_BNDS[:, 0] = 2.0
_BNDS[:, 1] = 3.0
_BNDS[:_N_BINS, 0] = _bounds[:-1]
_BNDS[0, 0] -= 1e-6
_BNDS[:_N_BINS, 1] = _bounds[1:]


def _ece_kernel(x_ref, lbl_ref, bnd_ref, out_ref, acc_ref):
    i = pl.program_id(0)

    @pl.when(i == 0)
    def _init():
        acc_ref[...] = jnp.zeros_like(acc_ref)

    x = x_ref[...]
    m = jnp.max(x, axis=1, keepdims=True)
    z = jnp.sum(jnp.exp(x - m), axis=1, keepdims=True)
    conf_col = 1.0 / z
    pred_col = jnp.argmax(x, axis=1, keepdims=True)

    conf = lax.transpose(conf_col, (1, 0))
    pred = lax.transpose(pred_col, (1, 0))
    lbl = lbl_ref[...].reshape(1, _BLK)
    hit = (pred == lbl).astype(jnp.float32)

    lo = bnd_ref[:, 0:1]
    up = bnd_ref[:, 1:2]
    maskf = ((conf > lo) & (conf <= up)).astype(jnp.float32)
    cnt = jnp.sum(maskf, axis=1, keepdims=True)
    sconf = jnp.sum(maskf * conf, axis=1, keepdims=True)
    sacc = jnp.sum(maskf * hit, axis=1, keepdims=True)
    acc_ref[:, 0:1] += cnt
    acc_ref[:, 1:2] += sconf
    acc_ref[:, 2:3] += sacc

    @pl.when(i == _GRID - 1)
    def _finish():
        tot = acc_ref[:, 0:1]
        sc = acc_ref[:, 1:2]
        sa = acc_ref[:, 2:3]
        safe = jnp.maximum(tot, 1.0)
        contrib = jnp.abs(sc / safe - sa / safe) * (tot / float(_N))
        contrib = jnp.where(tot > 0.0, contrib, 0.0)
        out_ref[...] = jnp.sum(contrib, axis=0, keepdims=True)


@jax.jit
def kernel(logits_input, labels_input):
    labels = labels_input.astype(jnp.int32).reshape(_GRID, 1, _BLK)
    out = pl.pallas_call(
        _ece_kernel,
        grid=(_GRID,),
        in_specs=[
            pl.BlockSpec((_BLK, _C), lambda i: (i, 0)),
            pl.BlockSpec((1, 1, _BLK), lambda i: (i, 0, 0)),
            pl.BlockSpec((16, 128), lambda i: (0, 0)),
        ],
        out_specs=pl.BlockSpec((1, 1), lambda i: (0, 0)),
        out_shape=jax.ShapeDtypeStruct((1, 1), jnp.float32),
        scratch_shapes=[pltpu.VMEM((16, 128), jnp.float32)],
        compiler_params=pltpu.CompilerParams(
            dimension_semantics=("arbitrary",),
        ),
    )(logits_input, labels, jnp.asarray(_BNDS))
    return out.reshape((1,))
